# Initial kernel scaffold; baseline (speedup 1.0000x reference)
import functools

import jax
import jax.numpy as jnp
from jax import lax
from jax.experimental import pallas as pl
from jax.experimental.pallas import tpu as pltpu
from jax.experimental.pallas import tpu_sc as plsc

B, L, NCH = 4096, 200, 3
VOCAB = 14
N_EMB = 64
D_FEAT = 10
D_EMB = NCH * N_EMB
D_OUT = D_EMB + D_FEAT
P = B * L

NC, NS = 2, 16
NW = NC * NS
PW = P // NW
CHUNK = 512
NCHUNK = PW // CHUNK
GB = 128
NGB = CHUNK // GB


def _sc_kernel_body(fsrc_hbm, se_hbm, ftab_hbm, out_hbm,
                    fidx_v, emb_v, se_v, gsem):
    wid = lax.axis_index("s") * NC + lax.axis_index("c")

    def chunk_body(i, carry):
        base = wid * PW + i * CHUNK
        pltpu.sync_copy(fsrc_hbm.at[pl.ds(base // GB, NGB)], fidx_v)
        copies = []
        for j in range(NGB):
            copies.append(
                pltpu.async_copy(ftab_hbm.at[fidx_v.at[j]],
                                 emb_v.at[pl.ds(j * GB, GB)], gsem))
        pltpu.sync_copy(se_hbm.at[pl.ds(base, CHUNK)], se_v)
        for c in copies:
            c.wait()
        pltpu.sync_copy(emb_v, out_hbm.at[pl.ds(base, CHUNK), pl.ds(0, D_EMB)])
        pltpu.sync_copy(se_v,
                        out_hbm.at[pl.ds(base, CHUNK), pl.ds(D_EMB, D_FEAT)])
        return carry

    lax.fori_loop(0, NCHUNK, chunk_body, 0)


@jax.jit
def _fused_embed(fsrc, se2, ftab):
    mesh = plsc.VectorSubcoreMesh(core_axis_name="c", subcore_axis_name="s")
    return pl.kernel(
        _sc_kernel_body,
        out_type=jax.ShapeDtypeStruct((P, D_OUT), jnp.float32),
        mesh=mesh,
        scratch_types=[
            pltpu.VMEM((NGB, GB), jnp.int32),
            pltpu.VMEM((CHUNK, D_EMB), jnp.float32),
            pltpu.VMEM((CHUNK, D_FEAT), jnp.float32),
            pltpu.SemaphoreType.DMA,
        ],
    )(fsrc, se2, ftab)


def kernel(src, se, emb):
    src = src.astype(jnp.int32)
    fsrc = (src[..., 0] * (VOCAB * VOCAB) + src[..., 1] * VOCAB
            + src[..., 2]).reshape(P // GB, GB)
    ids = jnp.arange(VOCAB * VOCAB * VOCAB, dtype=jnp.int32)
    ftab = jnp.concatenate(
        [emb[ids // (VOCAB * VOCAB)], emb[(ids // VOCAB) % VOCAB],
         emb[ids % VOCAB]], axis=1)
    se2 = se.reshape(P, D_FEAT)
    out = _fused_embed(fsrc, se2, ftab)
    return out.reshape(B, L, D_OUT)


# speedup vs baseline: 6.0954x; 6.0954x over previous
import functools

import jax
import jax.numpy as jnp
from jax import lax
from jax.experimental import pallas as pl
from jax.experimental.pallas import tpu as pltpu
from jax.experimental.pallas import tpu_sc as plsc

B, L, NCH = 4096, 200, 3
VOCAB = 14
N_EMB = 64
D_FEAT = 10
D_EMB = NCH * N_EMB
D_OUT = D_EMB + D_FEAT
P = B * L

NC, NS = 2, 16
NW = NC * NS
PW = P // NW
CHUNK = 512
NCHUNK = PW // CHUNK
GB = 128
NGB = CHUNK // GB


def _sc_kernel_body(fsrc_hbm, se_hbm, ftab_hbm, out_hbm,
                    fidx_v, emb_v, se_v, gsem):
    wid = lax.axis_index("s") * NC + lax.axis_index("c")

    def chunk_body(i, carry):
        base = wid * PW + i * CHUNK
        pltpu.sync_copy(fsrc_hbm.at[pl.ds(base // GB, NGB)], fidx_v)
        copies = []
        for j in range(NGB):
            copies.append(
                pltpu.async_copy(ftab_hbm.at[fidx_v.at[j]],
                                 emb_v.at[pl.ds(j * GB, GB)], gsem))
        pltpu.sync_copy(se_hbm.at[pl.ds(base, CHUNK)], se_v)
        for c in copies:
            c.wait()
        pltpu.sync_copy(emb_v, out_hbm.at[pl.ds(base, CHUNK), pl.ds(0, D_EMB)])
        pltpu.sync_copy(se_v,
                        out_hbm.at[pl.ds(base, CHUNK), pl.ds(D_EMB, D_FEAT)])
        return carry

    lax.fori_loop(0, NCHUNK, chunk_body, 0)


@jax.jit
def _fused_embed(fsrc, se2, ftab):
    mesh = plsc.VectorSubcoreMesh(core_axis_name="c", subcore_axis_name="s")
    return pl.kernel(
        _sc_kernel_body,
        out_type=jax.ShapeDtypeStruct((P, D_OUT), jnp.float32),
        mesh=mesh,
        scratch_types=[
            pltpu.VMEM((NGB, GB), jnp.int32),
            pltpu.VMEM((CHUNK, D_EMB), jnp.float32),
            pltpu.VMEM((CHUNK, D_FEAT), jnp.float32),
            pltpu.SemaphoreType.DMA,
        ],
        compiler_params=pltpu.CompilerParams(use_tc_tiling_on_sc=False),
    )(fsrc, se2, ftab)


def kernel(src, se, emb):
    src = src.astype(jnp.int32)
    fsrc = (src[..., 0] * (VOCAB * VOCAB) + src[..., 1] * VOCAB
            + src[..., 2]).reshape(P // GB, GB)
    ids = jnp.arange(VOCAB * VOCAB * VOCAB, dtype=jnp.int32)
    ftab = jnp.concatenate(
        [emb[ids // (VOCAB * VOCAB)], emb[(ids // VOCAB) % VOCAB],
         emb[ids % VOCAB]], axis=1)
    se2 = se.reshape(P, D_FEAT)
    out = _fused_embed(fsrc, se2, ftab)
    return out.reshape(B, L, D_OUT)


# speedup vs baseline: 6.1521x; 1.0093x over previous
import functools

import jax
import jax.numpy as jnp
from jax import lax
from jax.experimental import pallas as pl
from jax.experimental.pallas import tpu as pltpu
from jax.experimental.pallas import tpu_sc as plsc

B, L, NCH = 4096, 200, 3
VOCAB = 14
N_EMB = 64
D_FEAT = 10
D_EMB = NCH * N_EMB
D_OUT = D_EMB + D_FEAT
P = B * L

NC, NS = 2, 16
NW = NC * NS
PW = P // NW
CHUNK = 256
NCHUNK = PW // CHUNK
GB = 128
NGB = CHUNK // GB
NPAIR = NCHUNK // 2


def _sc_kernel_body(fsrc_hbm, se_hbm, ftab_hbm, out_hbm,
                    fidx_v, emb_v, se_v, gsem, wsem0, wsem1):
    wid = lax.axis_index("s") * NC + lax.axis_index("c")

    def pair_body(t, carry):
        for b, wsem in ((0, wsem0), (1, wsem1)):
            g = 2 * t + b
            base = wid * PW + g * CHUNK

            @pl.when(t >= 1)
            def _():
                pltpu.make_async_copy(
                    out_hbm.at[pl.ds(0, CHUNK), pl.ds(0, D_EMB)],
                    emb_v.at[b], wsem).wait()
                pltpu.make_async_copy(
                    out_hbm.at[pl.ds(0, CHUNK), pl.ds(D_EMB, D_FEAT)],
                    se_v.at[b], wsem).wait()

            pltpu.sync_copy(fsrc_hbm.at[pl.ds(base // GB, NGB)],
                            fidx_v.at[b])
            copies = []
            for j in range(NGB):
                copies.append(
                    pltpu.async_copy(ftab_hbm.at[fidx_v.at[b, j]],
                                     emb_v.at[b, pl.ds(j * GB, GB)], gsem))
            copies.append(
                pltpu.async_copy(se_hbm.at[pl.ds(base, CHUNK)],
                                 se_v.at[b], gsem))
            for c in copies:
                c.wait()
            pltpu.async_copy(emb_v.at[b],
                             out_hbm.at[pl.ds(base, CHUNK), pl.ds(0, D_EMB)],
                             wsem)
            pltpu.async_copy(se_v.at[b],
                             out_hbm.at[pl.ds(base, CHUNK),
                                        pl.ds(D_EMB, D_FEAT)],
                             wsem)
        return carry

    lax.fori_loop(0, NPAIR, pair_body, 0)
    for b, wsem in ((0, wsem0), (1, wsem1)):
        pltpu.make_async_copy(out_hbm.at[pl.ds(0, CHUNK), pl.ds(0, D_EMB)],
                              emb_v.at[b], wsem).wait()
        pltpu.make_async_copy(out_hbm.at[pl.ds(0, CHUNK),
                                         pl.ds(D_EMB, D_FEAT)],
                              se_v.at[b], wsem).wait()


@jax.jit
def _fused_embed(fsrc, se2, ftab):
    mesh = plsc.VectorSubcoreMesh(core_axis_name="c", subcore_axis_name="s")
    return pl.kernel(
        _sc_kernel_body,
        out_type=jax.ShapeDtypeStruct((P, D_OUT), jnp.float32),
        mesh=mesh,
        scratch_types=[
            pltpu.VMEM((2, NGB, GB), jnp.int32),
            pltpu.VMEM((2, CHUNK, D_EMB), jnp.float32),
            pltpu.VMEM((2, CHUNK, D_FEAT), jnp.float32),
            pltpu.SemaphoreType.DMA,
            pltpu.SemaphoreType.DMA,
            pltpu.SemaphoreType.DMA,
        ],
        compiler_params=pltpu.CompilerParams(use_tc_tiling_on_sc=False),
    )(fsrc, se2, ftab)


def kernel(src, se, emb):
    src = src.astype(jnp.int32)
    fsrc = (src[..., 0] * (VOCAB * VOCAB) + src[..., 1] * VOCAB
            + src[..., 2]).reshape(P // GB, GB)
    ids = jnp.arange(VOCAB * VOCAB * VOCAB, dtype=jnp.int32)
    ftab = jnp.concatenate(
        [emb[ids // (VOCAB * VOCAB)], emb[(ids // VOCAB) % VOCAB],
         emb[ids % VOCAB]], axis=1)
    se2 = se.reshape(P, D_FEAT)
    out = _fused_embed(fsrc, se2, ftab)
    return out.reshape(B, L, D_OUT)
